# Initial kernel scaffold; baseline (speedup 1.0000x reference)
import jax
import jax.numpy as jnp
from jax import lax
from jax.experimental import pallas as pl

C_IN = 65
H1, H2, H3, C_OUT = 1024, 512, 256, 22
TOPK = 8
T_TILE = 512


def _mlp_topk_body(x_ref, w1_ref, b1_ref, w2_ref, b2_ref, w3_ref, b3_ref,
                   w4_ref, b4_ref, o_ref):
    x = x_ref[0]
    h = jnp.maximum(jnp.dot(w1_ref[...], x, preferred_element_type=jnp.float32)
                    + b1_ref[...], 0.0)
    h = jnp.maximum(jnp.dot(w2_ref[...], h, preferred_element_type=jnp.float32)
                    + b2_ref[...], 0.0)
    h = jnp.maximum(jnp.dot(w3_ref[...], h, preferred_element_type=jnp.float32)
                    + b3_ref[...], 0.0)
    z = (jnp.dot(w4_ref[...], h, preferred_element_type=jnp.float32)
         + b4_ref[...])

    rows = lax.broadcasted_iota(jnp.int32, (C_OUT, T_TILE), 0)
    rank = jnp.zeros((C_OUT, T_TILE), jnp.int32)
    for j in range(C_OUT):
        xj = jnp.broadcast_to(z[j:j + 1, :], (C_OUT, T_TILE))
        gt = xj > z
        ge = xj >= z
        beats = jnp.where(rows > j, ge, gt)
        rank = rank + beats.astype(jnp.int32)
    o_ref[0] = jnp.where(rank < TOPK, z, 0.0)


@jax.jit
def kernel(input, W1, b1, W2, b2, W3, b3, W4, b4):
    B, C, T = input.shape
    grid = (B, T // T_TILE)

    out = pl.pallas_call(
        _mlp_topk_body,
        grid=grid,
        in_specs=[
            pl.BlockSpec((1, C_IN, T_TILE), lambda b, t: (b, 0, t)),
            pl.BlockSpec((H1, C_IN), lambda b, t: (0, 0)),
            pl.BlockSpec((H1, 1), lambda b, t: (0, 0)),
            pl.BlockSpec((H2, H1), lambda b, t: (0, 0)),
            pl.BlockSpec((H2, 1), lambda b, t: (0, 0)),
            pl.BlockSpec((H3, H2), lambda b, t: (0, 0)),
            pl.BlockSpec((H3, 1), lambda b, t: (0, 0)),
            pl.BlockSpec((C_OUT, H3), lambda b, t: (0, 0)),
            pl.BlockSpec((C_OUT, 1), lambda b, t: (0, 0)),
        ],
        out_specs=pl.BlockSpec((1, C_OUT, T_TILE), lambda b, t: (b, 0, t)),
        out_shape=jax.ShapeDtypeStruct((B, C_OUT, T), jnp.float32),
    )(
        input,
        W1.T, b1.reshape(H1, 1),
        W2.T, b2.reshape(H2, 1),
        W3.T, b3.reshape(H3, 1),
        W4.T, b4.reshape(C_OUT, 1),
    )
    return out


# speedup vs baseline: 9.9226x; 9.9226x over previous
import jax
import jax.numpy as jnp
from jax import lax
from jax.experimental import pallas as pl

C_IN = 65
H1, H2, H3, C_OUT = 1024, 512, 256, 22
TOPK = 8
T_TILE = 512


def _mlp_topk_body(x_ref, w1_ref, b1_ref, w2_ref, b2_ref, w3_ref, b3_ref,
                   w4_ref, b4_ref, o_ref):
    x = x_ref[0]
    h = jnp.maximum(jnp.dot(w1_ref[...], x, preferred_element_type=jnp.float32)
                    + b1_ref[...], 0.0)
    h = jnp.maximum(jnp.dot(w2_ref[...], h, preferred_element_type=jnp.float32)
                    + b2_ref[...], 0.0)
    h = jnp.maximum(jnp.dot(w3_ref[...], h, preferred_element_type=jnp.float32)
                    + b3_ref[...], 0.0)
    z = (jnp.dot(w4_ref[...], h, preferred_element_type=jnp.float32)
         + b4_ref[...])

    rows = lax.broadcasted_iota(jnp.int32, (C_OUT, T_TILE), 0)
    rank = jnp.zeros((C_OUT, T_TILE), jnp.int32)
    for j in range(C_OUT):
        xj = jnp.broadcast_to(z[j:j + 1, :], (C_OUT, T_TILE))
        gt = (xj > z).astype(jnp.int32)
        ge = (xj >= z).astype(jnp.int32)
        rank = rank + jnp.where(rows > j, ge, gt)
    o_ref[0] = jnp.where(rank < TOPK, z, 0.0)


@jax.jit
def kernel(input, W1, b1, W2, b2, W3, b3, W4, b4):
    B, C, T = input.shape
    grid = (B, T // T_TILE)

    out = pl.pallas_call(
        _mlp_topk_body,
        grid=grid,
        in_specs=[
            pl.BlockSpec((1, C_IN, T_TILE), lambda b, t: (b, 0, t)),
            pl.BlockSpec((H1, C_IN), lambda b, t: (0, 0)),
            pl.BlockSpec((H1, 1), lambda b, t: (0, 0)),
            pl.BlockSpec((H2, H1), lambda b, t: (0, 0)),
            pl.BlockSpec((H2, 1), lambda b, t: (0, 0)),
            pl.BlockSpec((H3, H2), lambda b, t: (0, 0)),
            pl.BlockSpec((H3, 1), lambda b, t: (0, 0)),
            pl.BlockSpec((C_OUT, H3), lambda b, t: (0, 0)),
            pl.BlockSpec((C_OUT, 1), lambda b, t: (0, 0)),
        ],
        out_specs=pl.BlockSpec((1, C_OUT, T_TILE), lambda b, t: (b, 0, t)),
        out_shape=jax.ShapeDtypeStruct((B, C_OUT, T), jnp.float32),
    )(
        input,
        W1.T, b1.reshape(H1, 1),
        W2.T, b2.reshape(H2, 1),
        W3.T, b3.reshape(H3, 1),
        W4.T, b4.reshape(C_OUT, 1),
    )
    return out


# speedup vs baseline: 11.9337x; 1.2027x over previous
import jax
import jax.numpy as jnp
from jax import lax
from jax.experimental import pallas as pl

C_IN = 65
H1, H2, H3, C_OUT = 1024, 512, 256, 22
TOPK = 8
T_TILE = 1024


def _mlp_topk_body(x_ref, w1_ref, b1_ref, w2_ref, b2_ref, w3_ref, b3_ref,
                   w4_ref, b4_ref, o_ref):
    x = x_ref[0]
    h = jnp.maximum(jnp.dot(w1_ref[...], x, preferred_element_type=jnp.float32)
                    + b1_ref[...], 0.0)
    h = jnp.maximum(jnp.dot(w2_ref[...], h, preferred_element_type=jnp.float32)
                    + b2_ref[...], 0.0)
    h = jnp.maximum(jnp.dot(w3_ref[...], h, preferred_element_type=jnp.float32)
                    + b3_ref[...], 0.0)
    z = (jnp.dot(w4_ref[...], h, preferred_element_type=jnp.float32)
         + b4_ref[...])

    rows = lax.broadcasted_iota(jnp.int32, (C_OUT, T_TILE), 0)
    rank = jnp.zeros((C_OUT, T_TILE), jnp.int32)
    for j in range(C_OUT):
        xj = jnp.broadcast_to(z[j:j + 1, :], (C_OUT, T_TILE))
        gt = (xj > z).astype(jnp.int32)
        ge = (xj >= z).astype(jnp.int32)
        rank = rank + jnp.where(rows > j, ge, gt)
    o_ref[0] = jnp.where(rank < TOPK, z, 0.0)


@jax.jit
def kernel(input, W1, b1, W2, b2, W3, b3, W4, b4):
    B, C, T = input.shape
    grid = (B, T // T_TILE)

    out = pl.pallas_call(
        _mlp_topk_body,
        grid=grid,
        in_specs=[
            pl.BlockSpec((1, C_IN, T_TILE), lambda b, t: (b, 0, t)),
            pl.BlockSpec((H1, C_IN), lambda b, t: (0, 0)),
            pl.BlockSpec((H1, 1), lambda b, t: (0, 0)),
            pl.BlockSpec((H2, H1), lambda b, t: (0, 0)),
            pl.BlockSpec((H2, 1), lambda b, t: (0, 0)),
            pl.BlockSpec((H3, H2), lambda b, t: (0, 0)),
            pl.BlockSpec((H3, 1), lambda b, t: (0, 0)),
            pl.BlockSpec((C_OUT, H3), lambda b, t: (0, 0)),
            pl.BlockSpec((C_OUT, 1), lambda b, t: (0, 0)),
        ],
        out_specs=pl.BlockSpec((1, C_OUT, T_TILE), lambda b, t: (b, 0, t)),
        out_shape=jax.ShapeDtypeStruct((B, C_OUT, T), jnp.float32),
    )(
        input,
        W1.T, b1.reshape(H1, 1),
        W2.T, b2.reshape(H2, 1),
        W3.T, b3.reshape(H3, 1),
        W4.T, b4.reshape(C_OUT, 1),
    )
    return out


# speedup vs baseline: 12.3326x; 1.0334x over previous
import jax
import jax.numpy as jnp
from jax import lax
from jax.experimental import pallas as pl

C_IN = 65
H1, H2, H3, C_OUT = 1024, 512, 256, 22
TOPK = 8
T_TILE = 2048


def _mlp_topk_body(x_ref, w1_ref, b1_ref, w2_ref, b2_ref, w3_ref, b3_ref,
                   w4_ref, b4_ref, o_ref):
    x = x_ref[0]
    h = jnp.maximum(jnp.dot(w1_ref[...], x, preferred_element_type=jnp.float32)
                    + b1_ref[...], 0.0)
    h = jnp.maximum(jnp.dot(w2_ref[...], h, preferred_element_type=jnp.float32)
                    + b2_ref[...], 0.0)
    h = jnp.maximum(jnp.dot(w3_ref[...], h, preferred_element_type=jnp.float32)
                    + b3_ref[...], 0.0)
    z = (jnp.dot(w4_ref[...], h, preferred_element_type=jnp.float32)
         + b4_ref[...])

    rows = lax.broadcasted_iota(jnp.int32, (C_OUT, T_TILE), 0)
    rank = jnp.zeros((C_OUT, T_TILE), jnp.int32)
    for j in range(C_OUT):
        xj = jnp.broadcast_to(z[j:j + 1, :], (C_OUT, T_TILE))
        gt = (xj > z).astype(jnp.int32)
        ge = (xj >= z).astype(jnp.int32)
        rank = rank + jnp.where(rows > j, ge, gt)
    o_ref[0] = jnp.where(rank < TOPK, z, 0.0)


@jax.jit
def kernel(input, W1, b1, W2, b2, W3, b3, W4, b4):
    B, C, T = input.shape
    grid = (B, T // T_TILE)

    out = pl.pallas_call(
        _mlp_topk_body,
        grid=grid,
        in_specs=[
            pl.BlockSpec((1, C_IN, T_TILE), lambda b, t: (b, 0, t)),
            pl.BlockSpec((H1, C_IN), lambda b, t: (0, 0)),
            pl.BlockSpec((H1, 1), lambda b, t: (0, 0)),
            pl.BlockSpec((H2, H1), lambda b, t: (0, 0)),
            pl.BlockSpec((H2, 1), lambda b, t: (0, 0)),
            pl.BlockSpec((H3, H2), lambda b, t: (0, 0)),
            pl.BlockSpec((H3, 1), lambda b, t: (0, 0)),
            pl.BlockSpec((C_OUT, H3), lambda b, t: (0, 0)),
            pl.BlockSpec((C_OUT, 1), lambda b, t: (0, 0)),
        ],
        out_specs=pl.BlockSpec((1, C_OUT, T_TILE), lambda b, t: (b, 0, t)),
        out_shape=jax.ShapeDtypeStruct((B, C_OUT, T), jnp.float32),
    )(
        input,
        W1.T, b1.reshape(H1, 1),
        W2.T, b2.reshape(H2, 1),
        W3.T, b3.reshape(H3, 1),
        W4.T, b4.reshape(C_OUT, 1),
    )
    return out


# speedup vs baseline: 12.4024x; 1.0057x over previous
import jax
import jax.numpy as jnp
from jax import lax
from jax.experimental import pallas as pl

C_IN = 65
H1, H2, H3, C_OUT = 1024, 512, 256, 22
TOPK = 8
T_TILE = 2048


def _mlp_topk_body(x_ref, w1_ref, w2_ref, b2_ref, w3_ref, b3_ref,
                   w4_ref, b4_ref, o_ref):
    x = x_ref[0]
    pad = jnp.zeros((128 - C_IN - 1, T_TILE), jnp.float32)
    ones = jnp.ones((1, T_TILE), jnp.float32)
    xp = jnp.concatenate([x, ones, pad], axis=0)
    h = jnp.maximum(jnp.dot(w1_ref[...], xp, preferred_element_type=jnp.float32),
                    0.0)
    h = jnp.maximum(jnp.dot(w2_ref[...], h, preferred_element_type=jnp.float32)
                    + b2_ref[...], 0.0)
    h = jnp.maximum(jnp.dot(w3_ref[...], h, preferred_element_type=jnp.float32)
                    + b3_ref[...], 0.0)
    z = (jnp.dot(w4_ref[...], h, preferred_element_type=jnp.float32)
         + b4_ref[...])

    rows = lax.broadcasted_iota(jnp.int32, (C_OUT, T_TILE), 0)
    rank = jnp.zeros((C_OUT, T_TILE), jnp.int32)
    for j in range(C_OUT):
        xj = jnp.broadcast_to(z[j:j + 1, :], (C_OUT, T_TILE))
        gt = (xj > z).astype(jnp.int32)
        ge = (xj >= z).astype(jnp.int32)
        rank = rank + jnp.where(rows > j, ge, gt)
    o_ref[0] = jnp.where(rank < TOPK, z, 0.0)


@jax.jit
def kernel(input, W1, b1, W2, b2, W3, b3, W4, b4):
    B, C, T = input.shape
    grid = (B, T // T_TILE)

    out = pl.pallas_call(
        _mlp_topk_body,
        grid=grid,
        in_specs=[
            pl.BlockSpec((1, C_IN, T_TILE), lambda b, t: (b, 0, t)),
            pl.BlockSpec((H1, 128), lambda b, t: (0, 0)),
            pl.BlockSpec((H2, H1), lambda b, t: (0, 0)),
            pl.BlockSpec((H2, 1), lambda b, t: (0, 0)),
            pl.BlockSpec((H3, H2), lambda b, t: (0, 0)),
            pl.BlockSpec((H3, 1), lambda b, t: (0, 0)),
            pl.BlockSpec((C_OUT, H3), lambda b, t: (0, 0)),
            pl.BlockSpec((C_OUT, 1), lambda b, t: (0, 0)),
        ],
        out_specs=pl.BlockSpec((1, C_OUT, T_TILE), lambda b, t: (b, 0, t)),
        out_shape=jax.ShapeDtypeStruct((B, C_OUT, T), jnp.float32),
    )(
        input,
        jnp.concatenate(
            [W1.T, b1.reshape(H1, 1), jnp.zeros((H1, 128 - C_IN - 1),
                                                jnp.float32)], axis=1),
        W2.T, b2.reshape(H2, 1),
        W3.T, b3.reshape(H3, 1),
        W4.T, b4.reshape(C_OUT, 1),
    )
    return out


# speedup vs baseline: 12.4220x; 1.0016x over previous
import jax
import jax.numpy as jnp
from jax import lax
from jax.experimental import pallas as pl
from jax.experimental.pallas import tpu as pltpu

C_IN = 65
K1 = 128
H1, H2, H3, C_OUT = 1024, 512, 256, 22
TOPK = 8
T_TILE = 2048


def _mlp_topk_body(x_ref, w1_ref, w2_ref, b2_ref, w3_ref, b3_ref,
                   w4_ref, b4_ref, o_ref, z_scr):
    s = pl.program_id(0)
    nb = pl.num_programs(0) - 1

    @pl.when(s > 0)
    def _topk():
        z = z_scr[...]
        rows = lax.broadcasted_iota(jnp.int32, (C_OUT, T_TILE), 0)
        rank = jnp.zeros((C_OUT, T_TILE), jnp.int32)
        for j in range(C_OUT):
            xj = jnp.broadcast_to(z[j:j + 1, :], (C_OUT, T_TILE))
            gt = (xj > z).astype(jnp.int32)
            ge = (xj >= z).astype(jnp.int32)
            rank = rank + jnp.where(rows > j, ge, gt)
        o_ref[0] = jnp.where(rank < TOPK, z, 0.0)

    @pl.when(s < nb)
    def _mlp():
        x = x_ref[0]
        pad = jnp.zeros((K1 - C_IN - 1, T_TILE), jnp.float32)
        ones = jnp.ones((1, T_TILE), jnp.float32)
        xp = jnp.concatenate([x, ones, pad], axis=0)
        h = jnp.maximum(
            jnp.dot(w1_ref[...], xp, preferred_element_type=jnp.float32), 0.0)
        h = jnp.maximum(
            jnp.dot(w2_ref[...], h, preferred_element_type=jnp.float32)
            + b2_ref[...], 0.0)
        h = jnp.maximum(
            jnp.dot(w3_ref[...], h, preferred_element_type=jnp.float32)
            + b3_ref[...], 0.0)
        z_scr[...] = (jnp.dot(w4_ref[...], h,
                              preferred_element_type=jnp.float32)
                      + b4_ref[...])


@jax.jit
def kernel(input, W1, b1, W2, b2, W3, b3, W4, b4):
    B, C, T = input.shape
    nt = T // T_TILE
    nb = B * nt
    grid = (nb + 1,)

    def x_map(s):
        sc = jnp.minimum(s, nb - 1)
        return (sc // nt, 0, sc % nt)

    def o_map(s):
        sp = jnp.maximum(s - 1, 0)
        return (sp // nt, 0, sp % nt)

    out = pl.pallas_call(
        _mlp_topk_body,
        grid=grid,
        in_specs=[
            pl.BlockSpec((1, C_IN, T_TILE), x_map),
            pl.BlockSpec((H1, K1), lambda s: (0, 0)),
            pl.BlockSpec((H2, H1), lambda s: (0, 0)),
            pl.BlockSpec((H2, 1), lambda s: (0, 0)),
            pl.BlockSpec((H3, H2), lambda s: (0, 0)),
            pl.BlockSpec((H3, 1), lambda s: (0, 0)),
            pl.BlockSpec((C_OUT, H3), lambda s: (0, 0)),
            pl.BlockSpec((C_OUT, 1), lambda s: (0, 0)),
        ],
        out_specs=pl.BlockSpec((1, C_OUT, T_TILE), o_map),
        out_shape=jax.ShapeDtypeStruct((B, C_OUT, T), jnp.float32),
        scratch_shapes=[pltpu.VMEM((C_OUT, T_TILE), jnp.float32)],
    )(
        input,
        jnp.concatenate(
            [W1.T, b1.reshape(H1, 1), jnp.zeros((H1, K1 - C_IN - 1),
                                                jnp.float32)], axis=1),
        W2.T, b2.reshape(H2, 1),
        W3.T, b3.reshape(H3, 1),
        W4.T, b4.reshape(C_OUT, 1),
    )
    return out
